# baseline (speedup 1.0000x reference)
import jax
import jax.numpy as jnp
import numpy as np
from jax import lax
from jax.experimental import pallas as pl
from jax.experimental.pallas import tpu as pltpu
from jax.experimental.pallas import tpu_sc as plsc

N = 10000
NP = 10240
D = 128
E = 320000

NC = 2
NS = 16
NW = NC * NS
CHUNK = 128
CPT = 80
EP = NW * CPT * CHUNK
ROWS_PER_TILE = NP // NS
BM = 1024


K0 = 48
K1 = 112
KMAX = max(K0, K1)
KMAXH = KMAX // 2
NCHUNKS = NS * (K0 + K1)
RCHUNKS = E // CHUNK


def _sc_agg_body(y_hbm, src_hbm, dst_hbm, zeros_hbm, zdeg_hbm,
                 out_acc, out_deg,
                 srcv, dstv, rows0, rows1, ones_v, acc_sh, deg_sh,
                 g0, g1, s0, s1, dsem):
    c = lax.axis_index("c")
    s = lax.axis_index("s")
    tid = c * NS + s
    r0 = s * ROWS_PER_TILE
    rows = (rows0, rows1)
    gsem = (g0, g1)
    ssem = (s0, s1)

    pltpu.sync_copy(zeros_hbm, acc_sh.at[pl.ds(r0, ROWS_PER_TILE)])
    pltpu.sync_copy(zdeg_hbm, deg_sh.at[pl.ds(r0, ROWS_PER_TILE)])
    for i in range(CHUNK // 16):
        ones_v[pl.ds(i * 16, 16)] = jnp.ones((16,), jnp.float32)
    plsc.subcore_barrier()

    def run_half(nchunks):
        pltpu.async_copy(y_hbm.at[srcv.at[0]], rows0, g0)
        pltpu.async_copy(y_hbm.at[srcv.at[1]], rows1, g1)

        def pair(p, carry):
            for b in range(2):
                j = 2 * p + b
                pltpu.make_async_copy(y_hbm.at[srcv.at[j]], rows[b],
                                      gsem[b]).wait()
                pltpu.async_copy(rows[b], acc_sh.at[dstv.at[j]], ssem[b],
                                 add=True)
                pltpu.async_copy(ones_v, deg_sh.at[dstv.at[j]], dsem,
                                 add=True)
                pltpu.make_async_copy(rows[b], acc_sh.at[dstv.at[j]],
                                      ssem[b]).wait()

                @pl.when(j + 2 < nchunks)
                def _():
                    pltpu.async_copy(y_hbm.at[srcv.at[j + 2]], rows[b],
                                     gsem[b])
            return carry

        lax.fori_loop(0, nchunks // 2, pair, 0)

        def drain(j, carry):
            pltpu.make_async_copy(ones_v, deg_sh.at[dstv.at[0]], dsem).wait()
            return carry

        lax.fori_loop(0, nchunks, drain, 0)

    for h in range(2):
        pltpu.sync_copy(src_hbm.at[tid, h], srcv)
        pltpu.sync_copy(dst_hbm.at[tid, h], dstv)

        @pl.when(c == 0)
        def _():
            run_half(K0 // 2)

        @pl.when(c == 1)
        def _():
            run_half(K1 // 2)

    plsc.subcore_barrier()
    pltpu.sync_copy(acc_sh.at[pl.ds(r0, ROWS_PER_TILE)], out_acc.at[c, s])
    pltpu.sync_copy(deg_sh.at[pl.ds(r0, ROWS_PER_TILE)],
                    out_deg.at[pl.ds(tid * ROWS_PER_TILE, ROWS_PER_TILE)])


_sc_agg = pl.kernel(
    _sc_agg_body,
    out_type=(jax.ShapeDtypeStruct((NC, NS, ROWS_PER_TILE, D), jnp.float32),
              jax.ShapeDtypeStruct((NC * NP,), jnp.float32)),
    mesh=plsc.VectorSubcoreMesh(core_axis_name="c", subcore_axis_name="s"),
    scratch_types=[
        pltpu.VMEM((KMAXH, CHUNK), jnp.int32),
        pltpu.VMEM((KMAXH, CHUNK), jnp.int32),
        pltpu.VMEM((CHUNK, D), jnp.float32),
        pltpu.VMEM((CHUNK, D), jnp.float32),
        pltpu.VMEM((CHUNK,), jnp.float32),
        pltpu.VMEM_SHARED((NP, D), jnp.float32),
        pltpu.VMEM_SHARED((NP,), jnp.float32),
        pltpu.SemaphoreType.DMA,
        pltpu.SemaphoreType.DMA,
        pltpu.SemaphoreType.DMA,
        pltpu.SemaphoreType.DMA,
        pltpu.SemaphoreType.DMA,
    ],
)


def _tc_pre_body(x_ref, ws_ref, wn_ref, b_ref, xs_out, y_out):
    x = x_ref[...]
    xs_out[...] = jnp.dot(x, ws_ref[...],
                          preferred_element_type=jnp.float32) + b_ref[...]
    y_out[...] = jnp.dot(x, wn_ref[...], preferred_element_type=jnp.float32)


def _tc_mid_body(xs_ref, parts_ref, degp_ref, ws_ref, wn_ref, b_ref,
                 hs_out, y_out):
    agg = parts_ref[0] + parts_ref[1]
    deg = jnp.maximum(degp_ref[0] + degp_ref[1], 1.0)
    h = jnp.maximum(xs_ref[...] + agg / deg, 0.0)
    hs_out[...] = jnp.dot(h, ws_ref[...],
                          preferred_element_type=jnp.float32) + b_ref[...]
    y_out[...] = jnp.dot(h, wn_ref[...], preferred_element_type=jnp.float32)


def _tc_post_body(hs_ref, parts_ref, degp_ref, out_ref):
    agg = parts_ref[0] + parts_ref[1]
    deg = jnp.maximum(degp_ref[0] + degp_ref[1], 1.0)
    z = hs_ref[...] + agg / deg
    m = jnp.max(z, axis=-1, keepdims=True)
    lse = jnp.log(jnp.sum(jnp.exp(z - m), axis=-1, keepdims=True))
    out_ref[...] = z - m - lse


_row_spec = pl.BlockSpec((BM, D), lambda i: (i, 0))
_mat_spec = pl.BlockSpec((D, D), lambda i: (0, 0))
_bias_spec = pl.BlockSpec((1, D), lambda i: (0, 0))
_parts_spec = pl.BlockSpec((NC, BM, D), lambda i: (0, i, 0))
_degp_spec = pl.BlockSpec((NC, BM, 1), lambda i: (0, i, 0))

_tc_pre = pl.pallas_call(
    _tc_pre_body,
    grid=(NP // BM,),
    in_specs=[_row_spec, _mat_spec, _mat_spec, _bias_spec],
    out_specs=(_row_spec, _row_spec),
    out_shape=(jax.ShapeDtypeStruct((NP, D), jnp.float32),
               jax.ShapeDtypeStruct((NP, D), jnp.float32)),
)

_tc_mid = pl.pallas_call(
    _tc_mid_body,
    grid=(NP // BM,),
    in_specs=[_row_spec, _parts_spec, _degp_spec,
              _mat_spec, _mat_spec, _bias_spec],
    out_specs=(_row_spec, _row_spec),
    out_shape=(jax.ShapeDtypeStruct((NP, D), jnp.float32),
               jax.ShapeDtypeStruct((NP, D), jnp.float32)),
)

_tc_post = pl.pallas_call(
    _tc_post_body,
    grid=(NP // BM,),
    in_specs=[_row_spec, _parts_spec, _degp_spec],
    out_specs=_row_spec,
    out_shape=jax.ShapeDtypeStruct((NP, D), jnp.float32),
)


def _make_row_map():
    m = np.zeros((NW, 2, KMAXH), np.int32)
    pos = 0
    for t in range(NW):
        kh = (K0 if t < NS else K1) // 2
        for h in range(2):
            m[t, h, :kh] = np.arange(pos, pos + kh, dtype=np.int32)
            pos += kh
    assert pos == NCHUNKS
    return m


_ROW_MAP = _make_row_map()


def kernel(x, W_self0, W_neigh0, b0, W_self1, W_neigh1, b1,
           edge_index1, edge_index2):
    npad = (NCHUNKS - RCHUNKS) * CHUNK
    pad_src = jnp.zeros((npad,), jnp.int32)
    pad_dst = N + (jnp.arange(npad, dtype=jnp.int32) % (NP - N))
    src1 = jnp.concatenate([edge_index1[0], pad_src]).reshape(NCHUNKS, CHUNK)
    dst1 = jnp.concatenate([edge_index1[1], pad_dst]).reshape(NCHUNKS, CHUNK)
    src2 = jnp.concatenate([edge_index2[0], pad_src]).reshape(NCHUNKS, CHUNK)
    dst2 = jnp.concatenate([edge_index2[1], pad_dst]).reshape(NCHUNKS, CHUNK)
    src1, dst1 = src1[_ROW_MAP], dst1[_ROW_MAP]
    src2, dst2 = src2[_ROW_MAP], dst2[_ROW_MAP]
    zeros = jnp.zeros((ROWS_PER_TILE, D), jnp.float32)
    zdeg = jnp.zeros((ROWS_PER_TILE,), jnp.float32)
    x_p = jnp.pad(x, ((0, NP - N), (0, 0)))

    xs0, y0 = _tc_pre(x_p, W_self0, W_neigh0, b0.reshape(1, D))
    parts1, degp1 = _sc_agg(y0, src1, dst1, zeros, zdeg)
    parts1 = parts1.reshape(NC, NP, D)
    degp1 = degp1.reshape(NC, NP, 1)
    hs1, y1 = _tc_mid(xs0, parts1, degp1, W_self1, W_neigh1, b1.reshape(1, D))
    parts2, degp2 = _sc_agg(y1, src2, dst2, zeros, zdeg)
    parts2 = parts2.reshape(NC, NP, D)
    degp2 = degp2.reshape(NC, NP, 1)
    return _tc_post(hs1, parts2, degp2)[:N]


# speedup vs baseline: 1.0707x; 1.0707x over previous
import jax
import jax.numpy as jnp
import numpy as np
from jax import lax
from jax.experimental import pallas as pl
from jax.experimental.pallas import tpu as pltpu
from jax.experimental.pallas import tpu_sc as plsc

N = 10000
NP = 10240
D = 128
E = 320000

NC = 2
NS = 16
NW = NC * NS
CHUNK = 128
CPT = 80
EP = NW * CPT * CHUNK
ROWS_PER_TILE = NP // NS
BM = 1024


K0 = 112
K1 = 48
KMAX = max(K0, K1)
KMAXH = KMAX // 2
NCHUNKS = NS * (K0 + K1)
RCHUNKS = E // CHUNK


def _sc_agg_body(y_hbm, src_hbm, dst_hbm, zeros_hbm, zdeg_hbm,
                 out_acc, out_deg,
                 srcv, dstv, rows0, rows1, ones_v, acc_sh, deg_sh,
                 g0, g1, s0, s1, dsem):
    c = lax.axis_index("c")
    s = lax.axis_index("s")
    tid = c * NS + s
    r0 = s * ROWS_PER_TILE
    rows = (rows0, rows1)
    gsem = (g0, g1)
    ssem = (s0, s1)

    pltpu.sync_copy(zeros_hbm, acc_sh.at[pl.ds(r0, ROWS_PER_TILE)])
    pltpu.sync_copy(zdeg_hbm, deg_sh.at[pl.ds(r0, ROWS_PER_TILE)])
    for i in range(CHUNK // 16):
        ones_v[pl.ds(i * 16, 16)] = jnp.ones((16,), jnp.float32)
    plsc.subcore_barrier()

    def run_half(nchunks):
        pltpu.async_copy(y_hbm.at[srcv.at[0]], rows0, g0)
        pltpu.async_copy(y_hbm.at[srcv.at[1]], rows1, g1)

        def pair(p, carry):
            for b in range(2):
                j = 2 * p + b
                pltpu.make_async_copy(y_hbm.at[srcv.at[j]], rows[b],
                                      gsem[b]).wait()
                pltpu.async_copy(rows[b], acc_sh.at[dstv.at[j]], ssem[b],
                                 add=True)
                pltpu.async_copy(ones_v, deg_sh.at[dstv.at[j]], dsem,
                                 add=True)
                pltpu.make_async_copy(rows[b], acc_sh.at[dstv.at[j]],
                                      ssem[b]).wait()

                @pl.when(j + 2 < nchunks)
                def _():
                    pltpu.async_copy(y_hbm.at[srcv.at[j + 2]], rows[b],
                                     gsem[b])
            return carry

        lax.fori_loop(0, nchunks // 2, pair, 0)

        def drain(j, carry):
            pltpu.make_async_copy(ones_v, deg_sh.at[dstv.at[0]], dsem).wait()
            return carry

        lax.fori_loop(0, nchunks, drain, 0)

    for h in range(2):
        pltpu.sync_copy(src_hbm.at[tid, h], srcv)
        pltpu.sync_copy(dst_hbm.at[tid, h], dstv)

        @pl.when(c == 0)
        def _():
            run_half(K0 // 2)

        @pl.when(c == 1)
        def _():
            run_half(K1 // 2)

    plsc.subcore_barrier()
    pltpu.sync_copy(acc_sh.at[pl.ds(r0, ROWS_PER_TILE)], out_acc.at[c, s])
    pltpu.sync_copy(deg_sh.at[pl.ds(r0, ROWS_PER_TILE)],
                    out_deg.at[pl.ds(tid * ROWS_PER_TILE, ROWS_PER_TILE)])


_sc_agg = pl.kernel(
    _sc_agg_body,
    out_type=(jax.ShapeDtypeStruct((NC, NS, ROWS_PER_TILE, D), jnp.float32),
              jax.ShapeDtypeStruct((NC * NP,), jnp.float32)),
    mesh=plsc.VectorSubcoreMesh(core_axis_name="c", subcore_axis_name="s"),
    scratch_types=[
        pltpu.VMEM((KMAXH, CHUNK), jnp.int32),
        pltpu.VMEM((KMAXH, CHUNK), jnp.int32),
        pltpu.VMEM((CHUNK, D), jnp.float32),
        pltpu.VMEM((CHUNK, D), jnp.float32),
        pltpu.VMEM((CHUNK,), jnp.float32),
        pltpu.VMEM_SHARED((NP, D), jnp.float32),
        pltpu.VMEM_SHARED((NP,), jnp.float32),
        pltpu.SemaphoreType.DMA,
        pltpu.SemaphoreType.DMA,
        pltpu.SemaphoreType.DMA,
        pltpu.SemaphoreType.DMA,
        pltpu.SemaphoreType.DMA,
    ],
)


def _tc_pre_body(x_ref, ws_ref, wn_ref, b_ref, xs_out, y_out):
    x = x_ref[...]
    xs_out[...] = jnp.dot(x, ws_ref[...],
                          preferred_element_type=jnp.float32) + b_ref[...]
    y_out[...] = jnp.dot(x, wn_ref[...], preferred_element_type=jnp.float32)


def _tc_mid_body(xs_ref, parts_ref, degp_ref, ws_ref, wn_ref, b_ref,
                 hs_out, y_out):
    agg = parts_ref[0] + parts_ref[1]
    deg = jnp.maximum(degp_ref[0] + degp_ref[1], 1.0)
    h = jnp.maximum(xs_ref[...] + agg / deg, 0.0)
    hs_out[...] = jnp.dot(h, ws_ref[...],
                          preferred_element_type=jnp.float32) + b_ref[...]
    y_out[...] = jnp.dot(h, wn_ref[...], preferred_element_type=jnp.float32)


def _tc_post_body(hs_ref, parts_ref, degp_ref, out_ref):
    agg = parts_ref[0] + parts_ref[1]
    deg = jnp.maximum(degp_ref[0] + degp_ref[1], 1.0)
    z = hs_ref[...] + agg / deg
    m = jnp.max(z, axis=-1, keepdims=True)
    lse = jnp.log(jnp.sum(jnp.exp(z - m), axis=-1, keepdims=True))
    out_ref[...] = z - m - lse


_row_spec = pl.BlockSpec((BM, D), lambda i: (i, 0))
_mat_spec = pl.BlockSpec((D, D), lambda i: (0, 0))
_bias_spec = pl.BlockSpec((1, D), lambda i: (0, 0))
_parts_spec = pl.BlockSpec((NC, BM, D), lambda i: (0, i, 0))
_degp_spec = pl.BlockSpec((NC, BM, 1), lambda i: (0, i, 0))

_tc_pre = pl.pallas_call(
    _tc_pre_body,
    grid=(NP // BM,),
    in_specs=[_row_spec, _mat_spec, _mat_spec, _bias_spec],
    out_specs=(_row_spec, _row_spec),
    out_shape=(jax.ShapeDtypeStruct((NP, D), jnp.float32),
               jax.ShapeDtypeStruct((NP, D), jnp.float32)),
)

_tc_mid = pl.pallas_call(
    _tc_mid_body,
    grid=(NP // BM,),
    in_specs=[_row_spec, _parts_spec, _degp_spec,
              _mat_spec, _mat_spec, _bias_spec],
    out_specs=(_row_spec, _row_spec),
    out_shape=(jax.ShapeDtypeStruct((NP, D), jnp.float32),
               jax.ShapeDtypeStruct((NP, D), jnp.float32)),
)

_tc_post = pl.pallas_call(
    _tc_post_body,
    grid=(NP // BM,),
    in_specs=[_row_spec, _parts_spec, _degp_spec],
    out_specs=_row_spec,
    out_shape=jax.ShapeDtypeStruct((NP, D), jnp.float32),
)


def _make_row_map():
    m = np.zeros((NW, 2, KMAXH), np.int32)
    pos = 0
    for t in range(NW):
        kh = (K0 if t < NS else K1) // 2
        for h in range(2):
            m[t, h, :kh] = np.arange(pos, pos + kh, dtype=np.int32)
            pos += kh
    assert pos == NCHUNKS
    return m


_ROW_MAP = _make_row_map()


def kernel(x, W_self0, W_neigh0, b0, W_self1, W_neigh1, b1,
           edge_index1, edge_index2):
    npad = (NCHUNKS - RCHUNKS) * CHUNK
    pad_src = jnp.zeros((npad,), jnp.int32)
    pad_dst = N + (jnp.arange(npad, dtype=jnp.int32) % (NP - N))
    src1 = jnp.concatenate([edge_index1[0], pad_src]).reshape(NCHUNKS, CHUNK)
    dst1 = jnp.concatenate([edge_index1[1], pad_dst]).reshape(NCHUNKS, CHUNK)
    src2 = jnp.concatenate([edge_index2[0], pad_src]).reshape(NCHUNKS, CHUNK)
    dst2 = jnp.concatenate([edge_index2[1], pad_dst]).reshape(NCHUNKS, CHUNK)
    src1, dst1 = src1[_ROW_MAP], dst1[_ROW_MAP]
    src2, dst2 = src2[_ROW_MAP], dst2[_ROW_MAP]
    zeros = jnp.zeros((ROWS_PER_TILE, D), jnp.float32)
    zdeg = jnp.zeros((ROWS_PER_TILE,), jnp.float32)
    x_p = jnp.pad(x, ((0, NP - N), (0, 0)))

    xs0, y0 = _tc_pre(x_p, W_self0, W_neigh0, b0.reshape(1, D))
    parts1, degp1 = _sc_agg(y0, src1, dst1, zeros, zdeg)
    parts1 = parts1.reshape(NC, NP, D)
    degp1 = degp1.reshape(NC, NP, 1)
    hs1, y1 = _tc_mid(xs0, parts1, degp1, W_self1, W_neigh1, b1.reshape(1, D))
    parts2, degp2 = _sc_agg(y1, src2, dst2, zeros, zdeg)
    parts2 = parts2.reshape(NC, NP, D)
    degp2 = degp2.reshape(NC, NP, 1)
    return _tc_post(hs1, parts2, degp2)[:N]


# speedup vs baseline: 1.1905x; 1.1120x over previous
import jax
import jax.numpy as jnp
import numpy as np
from jax import lax
from jax.experimental import pallas as pl
from jax.experimental.pallas import tpu as pltpu
from jax.experimental.pallas import tpu_sc as plsc

N = 10000
NP = 10240
D = 128
E = 320000

NC = 2
NS = 16
NW = NC * NS
CHUNK = 128
CPT = 80
EP = NW * CPT * CHUNK
ROWS_PER_TILE = NP // NS
BM = 1024


K0 = 80
K1 = 80
KMAX = max(K0, K1)
KMAXH = KMAX // 2
NCHUNKS = NS * (K0 + K1)
RCHUNKS = E // CHUNK


def _sc_agg_body(y_hbm, src_hbm, dst_hbm, zeros_hbm, zdeg_hbm,
                 out_acc, out_deg,
                 srcv, dstv, rows0, rows1, ones_v, acc_sh, deg_sh,
                 g0, g1, s0, s1, dsem):
    c = lax.axis_index("c")
    s = lax.axis_index("s")
    tid = c * NS + s
    r0 = s * ROWS_PER_TILE
    rows = (rows0, rows1)
    gsem = (g0, g1)
    ssem = (s0, s1)

    pltpu.sync_copy(zeros_hbm, acc_sh.at[pl.ds(r0, ROWS_PER_TILE)])
    pltpu.sync_copy(zdeg_hbm, deg_sh.at[pl.ds(r0, ROWS_PER_TILE)])
    for i in range(CHUNK // 16):
        ones_v[pl.ds(i * 16, 16)] = jnp.ones((16,), jnp.float32)
    plsc.subcore_barrier()

    def run_half(nchunks):
        pltpu.async_copy(y_hbm.at[srcv.at[0]], rows0, g0)
        pltpu.async_copy(y_hbm.at[srcv.at[1]], rows1, g1)

        def pair(p, carry):
            for b in range(2):
                j = 2 * p + b
                pltpu.make_async_copy(y_hbm.at[srcv.at[j]], rows[b],
                                      gsem[b]).wait()
                pltpu.async_copy(rows[b], acc_sh.at[dstv.at[j]], ssem[b],
                                 add=True)
                pltpu.async_copy(ones_v, deg_sh.at[dstv.at[j]], dsem,
                                 add=True)
                pltpu.make_async_copy(rows[b], acc_sh.at[dstv.at[j]],
                                      ssem[b]).wait()

                @pl.when(j + 2 < nchunks)
                def _():
                    pltpu.async_copy(y_hbm.at[srcv.at[j + 2]], rows[b],
                                     gsem[b])
            return carry

        lax.fori_loop(0, nchunks // 2, pair, 0)

        def drain(j, carry):
            pltpu.make_async_copy(ones_v, deg_sh.at[dstv.at[0]], dsem).wait()
            return carry

        lax.fori_loop(0, nchunks, drain, 0)

    for h in range(2):
        pltpu.sync_copy(src_hbm.at[tid, h], srcv)
        pltpu.sync_copy(dst_hbm.at[tid, h], dstv)

        @pl.when(c == 0)
        def _():
            run_half(K0 // 2)

        @pl.when(c == 1)
        def _():
            run_half(K1 // 2)

    plsc.subcore_barrier()
    pltpu.sync_copy(acc_sh.at[pl.ds(r0, ROWS_PER_TILE)], out_acc.at[c, s])
    pltpu.sync_copy(deg_sh.at[pl.ds(r0, ROWS_PER_TILE)],
                    out_deg.at[pl.ds(tid * ROWS_PER_TILE, ROWS_PER_TILE)])


_sc_agg = pl.kernel(
    _sc_agg_body,
    out_type=(jax.ShapeDtypeStruct((NC, NS, ROWS_PER_TILE, D), jnp.float32),
              jax.ShapeDtypeStruct((NC * NP,), jnp.float32)),
    mesh=plsc.VectorSubcoreMesh(core_axis_name="c", subcore_axis_name="s"),
    scratch_types=[
        pltpu.VMEM((KMAXH, CHUNK), jnp.int32),
        pltpu.VMEM((KMAXH, CHUNK), jnp.int32),
        pltpu.VMEM((CHUNK, D), jnp.float32),
        pltpu.VMEM((CHUNK, D), jnp.float32),
        pltpu.VMEM((CHUNK,), jnp.float32),
        pltpu.VMEM_SHARED((NP, D), jnp.float32),
        pltpu.VMEM_SHARED((NP,), jnp.float32),
        pltpu.SemaphoreType.DMA,
        pltpu.SemaphoreType.DMA,
        pltpu.SemaphoreType.DMA,
        pltpu.SemaphoreType.DMA,
        pltpu.SemaphoreType.DMA,
    ],
)


def _tc_pre_body(x_ref, ws_ref, wn_ref, b_ref, xs_out, y_out):
    x = x_ref[...]
    xs_out[...] = jnp.dot(x, ws_ref[...],
                          preferred_element_type=jnp.float32) + b_ref[...]
    y_out[...] = jnp.dot(x, wn_ref[...], preferred_element_type=jnp.float32)


def _tc_mid_body(xs_ref, parts_ref, degp_ref, ws_ref, wn_ref, b_ref,
                 hs_out, y_out):
    agg = parts_ref[0] + parts_ref[1]
    deg = jnp.maximum(degp_ref[0] + degp_ref[1], 1.0)
    h = jnp.maximum(xs_ref[...] + agg / deg, 0.0)
    hs_out[...] = jnp.dot(h, ws_ref[...],
                          preferred_element_type=jnp.float32) + b_ref[...]
    y_out[...] = jnp.dot(h, wn_ref[...], preferred_element_type=jnp.float32)


def _tc_post_body(hs_ref, parts_ref, degp_ref, out_ref):
    agg = parts_ref[0] + parts_ref[1]
    deg = jnp.maximum(degp_ref[0] + degp_ref[1], 1.0)
    z = hs_ref[...] + agg / deg
    m = jnp.max(z, axis=-1, keepdims=True)
    lse = jnp.log(jnp.sum(jnp.exp(z - m), axis=-1, keepdims=True))
    out_ref[...] = z - m - lse


_row_spec = pl.BlockSpec((BM, D), lambda i: (i, 0))
_mat_spec = pl.BlockSpec((D, D), lambda i: (0, 0))
_bias_spec = pl.BlockSpec((1, D), lambda i: (0, 0))
_parts_spec = pl.BlockSpec((NC, BM, D), lambda i: (0, i, 0))
_degp_spec = pl.BlockSpec((NC, BM, 1), lambda i: (0, i, 0))

_tc_pre = pl.pallas_call(
    _tc_pre_body,
    grid=(NP // BM,),
    in_specs=[_row_spec, _mat_spec, _mat_spec, _bias_spec],
    out_specs=(_row_spec, _row_spec),
    out_shape=(jax.ShapeDtypeStruct((NP, D), jnp.float32),
               jax.ShapeDtypeStruct((NP, D), jnp.float32)),
)

_tc_mid = pl.pallas_call(
    _tc_mid_body,
    grid=(NP // BM,),
    in_specs=[_row_spec, _parts_spec, _degp_spec,
              _mat_spec, _mat_spec, _bias_spec],
    out_specs=(_row_spec, _row_spec),
    out_shape=(jax.ShapeDtypeStruct((NP, D), jnp.float32),
               jax.ShapeDtypeStruct((NP, D), jnp.float32)),
)

_tc_post = pl.pallas_call(
    _tc_post_body,
    grid=(NP // BM,),
    in_specs=[_row_spec, _parts_spec, _degp_spec],
    out_specs=_row_spec,
    out_shape=jax.ShapeDtypeStruct((NP, D), jnp.float32),
)


def _make_row_map():
    m = np.zeros((NW, 2, KMAXH), np.int32)
    pos = 0
    for t in range(NW):
        kh = (K0 if t < NS else K1) // 2
        for h in range(2):
            m[t, h, :kh] = np.arange(pos, pos + kh, dtype=np.int32)
            pos += kh
    assert pos == NCHUNKS
    return m


_ROW_MAP = _make_row_map()


def kernel(x, W_self0, W_neigh0, b0, W_self1, W_neigh1, b1,
           edge_index1, edge_index2):
    npad = (NCHUNKS - RCHUNKS) * CHUNK
    pad_src = jnp.zeros((npad,), jnp.int32)
    pad_dst = N + (jnp.arange(npad, dtype=jnp.int32) % (NP - N))
    src1 = jnp.concatenate([edge_index1[0], pad_src]).reshape(NCHUNKS, CHUNK)
    dst1 = jnp.concatenate([edge_index1[1], pad_dst]).reshape(NCHUNKS, CHUNK)
    src2 = jnp.concatenate([edge_index2[0], pad_src]).reshape(NCHUNKS, CHUNK)
    dst2 = jnp.concatenate([edge_index2[1], pad_dst]).reshape(NCHUNKS, CHUNK)
    src1, dst1 = src1[_ROW_MAP], dst1[_ROW_MAP]
    src2, dst2 = src2[_ROW_MAP], dst2[_ROW_MAP]
    zeros = jnp.zeros((ROWS_PER_TILE, D), jnp.float32)
    zdeg = jnp.zeros((ROWS_PER_TILE,), jnp.float32)
    x_p = jnp.pad(x, ((0, NP - N), (0, 0)))

    xs0, y0 = _tc_pre(x_p, W_self0, W_neigh0, b0.reshape(1, D))
    parts1, degp1 = _sc_agg(y0, src1, dst1, zeros, zdeg)
    parts1 = parts1.reshape(NC, NP, D)
    degp1 = degp1.reshape(NC, NP, 1)
    hs1, y1 = _tc_mid(xs0, parts1, degp1, W_self1, W_neigh1, b1.reshape(1, D))
    parts2, degp2 = _sc_agg(y1, src2, dst2, zeros, zdeg)
    parts2 = parts2.reshape(NC, NP, D)
    degp2 = degp2.reshape(NC, NP, 1)
    return _tc_post(hs1, parts2, degp2)[:N]
